# baseline (speedup 1.0000x reference)
import functools

import jax
import jax.numpy as jnp
from jax import lax
from jax.experimental import pallas as pl
from jax.experimental.pallas import tpu as pltpu
from jax.experimental.pallas import tpu_sc as plsc

N = 10000
E = 160000
IN = 256
H = 512
OUT = 256
L = 4
EPS = 0.1

NP = 10240
C = 4
CW = 128
NC, NS = 2, 16
EB = 128
RPW = 88
RB = NS * RPW
EP = RB * EB
PAD_NODE = NP - 1
ROWS_PER_SUB = NP // NS
RBLK = 1280
GRID = NP // RBLK

_mesh = plsc.VectorSubcoreMesh(core_axis_name="c", subcore_axis_name="s")


def _zero_vmem(ref, nrows):
    def body(i, _):
        for j in range(ref.shape[1] // 16):
            ref[i, pl.ds(j * 16, 16)] = jnp.zeros((16,), jnp.float32)
        return 0

    lax.fori_loop(0, nrows, body, 0)


@functools.partial(
    pl.kernel,
    out_type=jax.ShapeDtypeStruct((NP, 16), jnp.float32),
    mesh=_mesh,
    scratch_types=[
        pltpu.VMEM((RPW, EB), jnp.int32),
        pltpu.VMEM((128, 16), jnp.float32),
        pltpu.VMEM((128, 16), jnp.float32),
        pltpu.VMEM_SHARED((NP, 16), jnp.float32),
    ],
    compiler_params=pltpu.CompilerParams(use_tc_tiling_on_sc=False),
)
def _sc_degree(dst_hbm, deg_out, dstv, onesv, zv, acc):
    cc = lax.axis_index("c")
    sid = lax.axis_index("s")

    @pl.when(cc == 0)
    def _():
        lane = lax.iota(jnp.int32, 16)
        e2 = jnp.where(lane == 2, 1.0, 0.0).astype(jnp.float32)

        def fill(i, _):
            onesv[i, :] = e2
            zv[i, :] = jnp.zeros((16,), jnp.float32)
            return 0

        lax.fori_loop(0, 128, fill, 0)
        for k in range(ROWS_PER_SUB // 128):
            pltpu.sync_copy(zv, acc.at[pl.ds(sid * ROWS_PER_SUB + k * 128, 128)])
        plsc.subcore_barrier()
        pltpu.sync_copy(dst_hbm.at[pl.ds(sid * RPW, RPW)], dstv)

        def batch(b, _):
            pltpu.sync_copy(onesv, acc.at[dstv.at[b]], add=True)
            return 0

        lax.fori_loop(0, RPW, batch, 0)
        plsc.subcore_barrier()
        for k in range(ROWS_PER_SUB // 128):
            sl = pl.ds(sid * ROWS_PER_SUB + k * 128, 128)
            pltpu.sync_copy(acc.at[sl], deg_out.at[sl])


BLK = 16
NBLK = RB // BLK
BW_N = NP // (NC * NS)
LB = 4352
FLUSH = 2048
WCAP = 128


@functools.partial(
    pl.kernel,
    out_type=[jax.ShapeDtypeStruct((NC * NS * EP,), jnp.int32),
              jax.ShapeDtypeStruct((NC * NS * 16,), jnp.int32)],
    mesh=_mesh,
    scratch_types=[
        pltpu.VMEM((2, BLK, EB), jnp.int32),
        pltpu.VMEM((2, BLK, EB), jnp.int32),
        pltpu.VMEM((LB,), jnp.int32),
        pltpu.VMEM((16,), jnp.int32),
        pltpu.SemaphoreType.DMA,
    ],
    compiler_params=pltpu.CompilerParams(use_tc_tiling_on_sc=False),
)
def _sc_bucket(src_hbm, dst_hbm, bp_out, cnt_out, sv, dv, lbuf, cb, sem):
    cc = lax.axis_index("c")
    sid = lax.axis_index("s")
    wid = sid * NC + cc
    lo = wid * BW_N
    ebase = wid * EP

    pltpu.async_copy(src_hbm.at[pl.ds(0, BLK)], sv.at[0], sem)
    pltpu.async_copy(dst_hbm.at[pl.ds(0, BLK)], dv.at[0], sem)
    pltpu.async_copy(src_hbm.at[pl.ds(BLK, BLK)], sv.at[1], sem)
    pltpu.async_copy(dst_hbm.at[pl.ds(BLK, BLK)], dv.at[1], sem)

    def block(blk, carry):
        fill, outrow = carry
        for par in range(2):
            @pl.when(lax.rem(blk, 2) == par)
            def _():
                pltpu.make_async_copy(
                    src_hbm.at[pl.ds(blk * BLK, BLK)], sv.at[par], sem).wait()
                pltpu.make_async_copy(
                    dst_hbm.at[pl.ds(blk * BLK, BLK)], dv.at[par], sem).wait()

        def row2(r, fill_):
            par2 = lax.rem(r, 2 * BLK) // BLK
            rr = lax.rem(r, BLK)

            def grp(g, f):
                s16 = sv[par2, rr, pl.ds(g * 16, 16)]
                d16 = dv[par2, rr, pl.ds(g * 16, 16)]
                m = (d16 >= lo) & (d16 < lo + BW_N)
                pair = (d16 << 14) | s16
                m32 = jnp.where(m, 1, 0)
                for k in range(16):
                    mk = m32[k]

                    @pl.when(mk == 1)
                    def _(pk=pair[k], f_=f):
                        lbuf[pl.ds(f_, 16)] = jnp.full((16,), pk, jnp.int32)

                    f = f + mk
                return f

            return lax.fori_loop(0, EB // 16, grp, fill_)

        fill = lax.fori_loop(blk * BLK, blk * BLK + BLK,
                             lambda r, f: row2(r, f), fill)

        def do_flush(args):
            f, orow = args
            pltpu.sync_copy(lbuf.at[pl.ds(0, FLUSH)],
                            bp_out.at[pl.ds(ebase + orow * EB, FLUSH)])

            def mv(k, _):
                lbuf[pl.ds(k * 16, 16)] = lbuf[pl.ds(FLUSH + k * 16, 16)]
                return 0

            lax.fori_loop(0, FLUSH // 16, mv, 0)
            return f - FLUSH, orow + FLUSH // EB

        fill, outrow = lax.cond(fill >= FLUSH, do_flush, lambda a: a,
                                (fill, outrow))
        for par in range(2):
            @pl.when((lax.rem(blk, 2) == par) & (blk + 2 < NBLK))
            def _():
                pltpu.async_copy(
                    src_hbm.at[pl.ds((blk + 2) * BLK, BLK)], sv.at[par], sem)
                pltpu.async_copy(
                    dst_hbm.at[pl.ds((blk + 2) * BLK, BLK)], dv.at[par], sem)
        return fill, outrow

    fill, outrow = lax.fori_loop(0, NBLK, block,
                                 (jnp.int32(0), jnp.int32(0)))

    total = outrow * EB + fill
    padpair = jnp.full((16,), ((PAD_NODE) << 14) | (PAD_NODE - 1), jnp.int32)
    for k in range(8):
        lbuf[pl.ds(fill + k * 16, 16)] = padpair

    nrows_left = (fill + EB - 1) // EB

    def fl(r, _):
        pltpu.sync_copy(lbuf.at[pl.ds(r * EB, EB)],
                        bp_out.at[pl.ds(ebase + (outrow + r) * EB, EB)])
        return 0

    lax.fori_loop(0, nrows_left, fl, 0)

    lane = lax.iota(jnp.int32, 16)
    cb[...] = jnp.where(lane == 0, total, 0)
    pltpu.sync_copy(cb, cnt_out.at[pl.ds(wid * 16, 16)])


@functools.partial(
    pl.kernel,
    out_type=[jax.ShapeDtypeStruct((NP, CW), jnp.float32) for _ in range(C)],
    mesh=_mesh,
    scratch_types=[
        pltpu.VMEM((3, EB), jnp.int32),
        pltpu.VMEM((3, EB), jnp.int32),
        pltpu.VMEM((3, EB), jnp.int32),
        pltpu.VMEM((3, EB), jnp.int32),
        pltpu.VMEM((3, EB), jnp.float32),
        pltpu.VMEM((3, EB), jnp.float32),
        pltpu.VMEM((3, EB), jnp.float32),
        pltpu.VMEM((3, EB), jnp.float32),
        pltpu.VMEM((3, EB, CW), jnp.float32),
        pltpu.VMEM((WCAP, EB), jnp.float32),
        pltpu.VMEM((BW_N + 8, CW), jnp.float32),
        pltpu.VMEM((16,), jnp.int32),
        pltpu.SemaphoreType.DMA,
        pltpu.SemaphoreType.DMA,
        pltpu.SemaphoreType.DMA,
    ],
    compiler_params=pltpu.CompilerParams(use_tc_tiling_on_sc=False),
)
def _sc_spmm(bp_hbm, cnt_hbm, al_hbm, ar_hbm, di_hbm,
             h0b, h1b, h2b, h3b, o0, o1, o2, o3,
             pb, srcv, dstv, dlv, als, ard, dis, did,
             rows, wc, lacc, cb, psem, gsem, wsem):
    cc = lax.axis_index("c")
    sid = lax.axis_index("s")
    wid = sid * NC + cc
    lo = wid * BW_N
    ebase = wid * EP

    pltpu.sync_copy(cnt_hbm.at[pl.ds(wid * 16, 16)], cb)
    cntw = cb[...][0]
    nb = (cntw + EB - 1) // EB

    def unpack(par):
        for g in range(EB // 16):
            sl = pl.ds(g * 16, 16)
            p16 = pb[par, sl]
            d16 = p16 >> 14
            s16 = p16 & 16383
            srcv[par, sl] = s16
            dstv[par, sl] = d16
            dlv[par, sl] = jnp.minimum(d16 - lo, BW_N)

    def fire_w(par):
        pltpu.async_copy(al_hbm.at[srcv.at[par]], als.at[par], wsem)
        pltpu.async_copy(ar_hbm.at[dstv.at[par]], ard.at[par], wsem)
        pltpu.async_copy(di_hbm.at[srcv.at[par]], dis.at[par], wsem)
        pltpu.async_copy(di_hbm.at[dstv.at[par]], did.at[par], wsem)

    def wait_w(par):
        pltpu.make_async_copy(al_hbm.at[srcv.at[par]], als.at[par], wsem).wait()
        pltpu.make_async_copy(ar_hbm.at[dstv.at[par]], ard.at[par], wsem).wait()
        pltpu.make_async_copy(di_hbm.at[srcv.at[par]], dis.at[par], wsem).wait()
        pltpu.make_async_copy(di_hbm.at[dstv.at[par]], did.at[par], wsem).wait()

    def wfresh(par, g):
        sl = pl.ds(g * 16, 16)
        a = als[par, sl] + ard[par, sl]
        e = jnp.exp(jnp.abs(a) * -2.0)
        t = (1.0 - e) / (1.0 + e)
        return dis[par, sl] * did[par, sl] * jnp.sign(a) * t

    def chunk(h_hbm, o_hbm, first):
        def zrow(i, _):
            for j in range(CW // 16):
                lacc[i, pl.ds(j * 16, 16)] = jnp.zeros((16,), jnp.float32)
            return 0

        lax.fori_loop(0, BW_N + 8, zrow, 0)

        def stage(j, par):
            pltpu.make_async_copy(bp_hbm.at[pl.ds(ebase + j * EB, EB)],
                                  pb.at[par], psem).wait()
            unpack(par)
            pltpu.async_copy(h_hbm.at[srcv.at[par]], rows.at[par], gsem)
            if first:
                fire_w(par)
            else:
                @pl.when(j >= WCAP)
                def _():
                    fire_w(par)

        @pl.when(nb > 0)
        def _():
            for j in range(3):
                @pl.when(j < nb)
                def _(j=j):
                    pltpu.async_copy(bp_hbm.at[pl.ds(ebase + j * EB, EB)],
                                     pb.at[j], psem)
            for j in range(2):
                @pl.when(j < nb)
                def _(j=j):
                    stage(j, j)

            def outer(b_, _):
                par0 = lax.rem(b_, 3)
                pre = b_ + 2
                pp = lax.rem(pre, 3)

                @pl.when(pre < nb)
                def _():
                    @pl.when(pre + 1 < nb)
                    def _():
                        pltpu.async_copy(
                            bp_hbm.at[pl.ds(ebase + (pre + 1) * EB, EB)],
                            pb.at[lax.rem(pre + 1, 3)], psem)
                    stage(pre, pp)

                pltpu.make_async_copy(h_hbm.at[srcv.at[par0]],
                                      rows.at[par0], gsem).wait()
                if first:
                    wait_w(par0)
                else:
                    @pl.when(b_ >= WCAP)
                    def _():
                        wait_w(par0)

                bcl = jnp.minimum(b_, WCAP - 1)

                def sgroup(g, _):
                    sl = pl.ds(g * 16, 16)
                    if first:
                        w16 = wfresh(par0, g)

                        @pl.when(b_ < WCAP)
                        def _():
                            wc[bcl, sl] = w16
                    else:
                        w16 = jnp.where(b_ < WCAP, wc[bcl, sl],
                                        wfresh(par0, g))
                    dl16 = dlv[par0, sl]
                    for k in range(16):
                        wi = w16[k]
                        dl = dl16[k]
                        i = g * 16 + k
                        for j in range(CW // 16):
                            cs = pl.ds(j * 16, 16)
                            plsc.addupdate(lacc.at[dl, cs],
                                           rows[par0, i, cs] * wi)
                    return 0

                lax.fori_loop(0, EB // 16, sgroup, 0)
                return 0

            lax.fori_loop(0, nb, outer, 0)

        pltpu.sync_copy(lacc.at[pl.ds(0, BW_N)], o_hbm.at[pl.ds(lo, BW_N)])

    for cix, (hh, oo) in enumerate(zip(
            (h0b, h1b, h2b, h3b), (o0, o1, o2, o3))):
        chunk(hh, oo, cix == 0)


def _dinv16(deg):
    return jnp.where(deg > 0, lax.rsqrt(jnp.maximum(deg, 1e-30)), 0.0)


def _tc_prep_body(x_ref, w_ref, b_ref, deg_ref, att_ref, *rest):
    hrefs, t_ref = rest[:C], rest[C]
    h = jnp.dot(x_ref[...], w_ref[...], preferred_element_type=jnp.float32)
    h = h + b_ref[...]
    for cix, hr in enumerate(hrefs):
        hr[...] = h[:, cix * CW:(cix + 1) * CW]
    t_ref[...] = (jnp.dot(h, att_ref[...], preferred_element_type=jnp.float32)
                  + _dinv16(deg_ref[...]))


def _tc_mid_body(*refs):
    ocs, pcs = refs[:C], refs[C:2 * C]
    att_ref, deg_ref = refs[2 * C], refs[2 * C + 1]
    hrefs, t_ref = refs[2 * C + 2:3 * C + 2], refs[3 * C + 2]
    hcs = []
    for o, p, hr in zip(ocs, pcs, hrefs):
        hc = o[...] + EPS * p[...]
        hr[...] = hc
        hcs.append(hc)
    h = jnp.concatenate(hcs, axis=1)
    t_ref[...] = (jnp.dot(h, att_ref[...], preferred_element_type=jnp.float32)
                  + _dinv16(deg_ref[...]))


def _tc_final_body(*refs):
    ocs, pcs = refs[:C], refs[C:2 * C]
    w_ref, b_ref, y_ref = refs[2 * C], refs[2 * C + 1], refs[2 * C + 2]
    h = jnp.concatenate(
        [o[...] + EPS * p[...] for o, p in zip(ocs, pcs)], axis=1)
    y_ref[...] = (jnp.dot(h, w_ref[...], preferred_element_type=jnp.float32)
                  + b_ref[...])


def _row_spec(d):
    return pl.BlockSpec((RBLK, d), lambda i: (i, 0))


def _full_spec(r, d):
    return pl.BlockSpec((r, d), lambda i: (0, 0))


_tc_prep = pl.pallas_call(
    _tc_prep_body,
    grid=(GRID,),
    in_specs=[_row_spec(IN), _full_spec(IN, H), _full_spec(1, H),
              _row_spec(16), _full_spec(H, 16)],
    out_specs=[_row_spec(CW)] * C + [_row_spec(16)],
    out_shape=[jax.ShapeDtypeStruct((NP, CW), jnp.float32) for _ in range(C)]
    + [jax.ShapeDtypeStruct((NP, 16), jnp.float32)],
)

_tc_mid = pl.pallas_call(
    _tc_mid_body,
    grid=(GRID,),
    in_specs=[_row_spec(CW)] * (2 * C) + [_full_spec(H, 16), _row_spec(16)],
    out_specs=[_row_spec(CW)] * C + [_row_spec(16)],
    out_shape=[jax.ShapeDtypeStruct((NP, CW), jnp.float32) for _ in range(C)]
    + [jax.ShapeDtypeStruct((NP, 16), jnp.float32)],
)

_tc_final = pl.pallas_call(
    _tc_final_body,
    grid=(GRID,),
    in_specs=[_row_spec(CW)] * (2 * C) + [_full_spec(H, OUT), _full_spec(1, OUT)],
    out_specs=_row_spec(OUT),
    out_shape=jax.ShapeDtypeStruct((NP, OUT), jnp.float32),
)


def kernel(x, edge_index, W_in, b_in, att_l, att_r, W_out, b_out):
    loop = jnp.arange(N, dtype=jnp.int32)
    src = jnp.concatenate([edge_index[0], loop])
    dst = jnp.concatenate([edge_index[1], loop])
    npad = EP - E - N
    pad_src = jnp.full((npad,), PAD_NODE - 1, dtype=jnp.int32)
    pad_dd = jnp.full((npad,), PAD_NODE, dtype=jnp.int32)
    pad_db = (jnp.arange(npad, dtype=jnp.int32) % (NC * NS)) * BW_N + (BW_N - 1)
    src2d = jnp.concatenate([src, pad_src]).reshape(RB, EB)
    dstd2d = jnp.concatenate([dst, pad_dd]).reshape(RB, EB)
    dstb2d = jnp.concatenate([dst, pad_db]).reshape(RB, EB)
    xp = jnp.pad(x, ((0, NP - N), (0, 0)))
    att_big = [
        jnp.zeros((H, 16), jnp.float32)
        .at[:, 0].set(att_l[l]).at[:, 1].set(att_r[l])
        for l in range(L)
    ]
    b_in2 = b_in[None, :]
    b_out2 = b_out[None, :]

    deg16 = _sc_degree(dstd2d)
    bpair, cnt = _sc_bucket(src2d, dstb2d)
    *h0c, t = _tc_prep(xp, W_in, b_in2, deg16, att_big[0])
    hc = list(h0c)
    oc = None
    for l in range(L):
        al1, ar1, di1 = t[:, 0], t[:, 1], t[:, 2]
        oc = _sc_spmm(bpair, cnt, al1, ar1, di1, *hc)
        if l + 1 < L:
            *hc, t = _tc_mid(*oc, *h0c, att_big[l + 1], deg16)
    y = _tc_final(*oc, *h0c, W_out, b_out2)
    return y[:N]


# speedup vs baseline: 1.3868x; 1.3868x over previous
import functools

import jax
import jax.numpy as jnp
from jax import lax
from jax.experimental import pallas as pl
from jax.experimental.pallas import tpu as pltpu
from jax.experimental.pallas import tpu_sc as plsc

N = 10000
E = 160000
IN = 256
H = 512
OUT = 256
L = 4
EPS = 0.1

NP = 10240
C = 8
CW = 64
NC, NS = 2, 16
EB = 128
RPW = 88
RB = NS * RPW
EP = RB * EB
PAD_NODE = NP - 1
ROWS_PER_SUB = NP // NS
RBLK = 1280
GRID = NP // RBLK

_mesh = plsc.VectorSubcoreMesh(core_axis_name="c", subcore_axis_name="s")


def _zero_vmem(ref, nrows):
    def body(i, _):
        for j in range(ref.shape[1] // 16):
            ref[i, pl.ds(j * 16, 16)] = jnp.zeros((16,), jnp.float32)
        return 0

    lax.fori_loop(0, nrows, body, 0)


@functools.partial(
    pl.kernel,
    out_type=jax.ShapeDtypeStruct((NP, 16), jnp.float32),
    mesh=_mesh,
    scratch_types=[
        pltpu.VMEM((RPW, EB), jnp.int32),
        pltpu.VMEM((128, 16), jnp.float32),
        pltpu.VMEM((128, 16), jnp.float32),
        pltpu.VMEM_SHARED((NP, 16), jnp.float32),
    ],
    compiler_params=pltpu.CompilerParams(use_tc_tiling_on_sc=False),
)
def _sc_degree(dst_hbm, deg_out, dstv, onesv, zv, acc):
    cc = lax.axis_index("c")
    sid = lax.axis_index("s")

    @pl.when(cc == 0)
    def _():
        lane = lax.iota(jnp.int32, 16)
        e2 = jnp.where(lane == 2, 1.0, 0.0).astype(jnp.float32)

        def fill(i, _):
            onesv[i, :] = e2
            zv[i, :] = jnp.zeros((16,), jnp.float32)
            return 0

        lax.fori_loop(0, 128, fill, 0)
        for k in range(ROWS_PER_SUB // 128):
            pltpu.sync_copy(zv, acc.at[pl.ds(sid * ROWS_PER_SUB + k * 128, 128)])
        plsc.subcore_barrier()
        pltpu.sync_copy(dst_hbm.at[pl.ds(sid * RPW, RPW)], dstv)

        def batch(b, _):
            pltpu.sync_copy(onesv, acc.at[dstv.at[b]], add=True)
            return 0

        lax.fori_loop(0, RPW, batch, 0)
        plsc.subcore_barrier()
        for k in range(ROWS_PER_SUB // 128):
            sl = pl.ds(sid * ROWS_PER_SUB + k * 128, 128)
            pltpu.sync_copy(acc.at[sl], deg_out.at[sl])


@functools.partial(
    pl.kernel,
    out_type=[jax.ShapeDtypeStruct((NP, CW), jnp.float32) for _ in range(C)],
    mesh=_mesh,
    scratch_types=[
        pltpu.VMEM((RPW, EB), jnp.int32),
        pltpu.VMEM((RPW, EB), jnp.int32),
        pltpu.VMEM((RPW, EB), jnp.float32),
        pltpu.VMEM((2, EB), jnp.float32),
        pltpu.VMEM((2, EB), jnp.float32),
        pltpu.VMEM((2, EB), jnp.float32),
        pltpu.VMEM((2, EB), jnp.float32),
        pltpu.VMEM((2, EB, CW), jnp.float32),
        pltpu.VMEM((128, CW), jnp.float32),
        pltpu.VMEM_SHARED((NP, CW), jnp.float32),
        pltpu.SemaphoreType.DMA,
        pltpu.SemaphoreType.DMA,
        pltpu.SemaphoreType.DMA,
        pltpu.SemaphoreType.DMA,
    ],
    compiler_params=pltpu.CompilerParams(use_tc_tiling_on_sc=False),
)
def _sc_spmm(src_hbm, dst_hbm, al_hbm, ar_hbm, di_hbm,
             h0b, h1b, h2b, h3b, h4b, h5b, h6b, h7b,
             o0, o1, o2, o3, o4, o5, o6, o7,
             srcv, dstv, wv, als, ard, dis, did,
             rows, zbuf, acc, gsem, ssem, wsem, rsem):
    cc = lax.axis_index("c")
    sid = lax.axis_index("s")
    base = sid * RPW
    pltpu.sync_copy(src_hbm.at[pl.ds(base, RPW)], srcv)
    pltpu.sync_copy(dst_hbm.at[pl.ds(base, RPW)], dstv)
    _zero_vmem(zbuf, 128)

    def fire_w(b, p):
        pltpu.async_copy(al_hbm.at[srcv.at[b]], als.at[p], wsem)
        pltpu.async_copy(ar_hbm.at[dstv.at[b]], ard.at[p], wsem)
        pltpu.async_copy(di_hbm.at[srcv.at[b]], dis.at[p], wsem)
        pltpu.async_copy(di_hbm.at[dstv.at[b]], did.at[p], wsem)

    def wait_w(b, p):
        pltpu.make_async_copy(al_hbm.at[srcv.at[b]], als.at[p], wsem).wait()
        pltpu.make_async_copy(ar_hbm.at[dstv.at[b]], ard.at[p], wsem).wait()
        pltpu.make_async_copy(di_hbm.at[srcv.at[b]], dis.at[p], wsem).wait()
        pltpu.make_async_copy(di_hbm.at[dstv.at[b]], did.at[p], wsem).wait()

    def chunk(h_hbm, o_hbm, first):
        for k in range(ROWS_PER_SUB // 128):
            pltpu.sync_copy(zbuf, acc.at[pl.ds(sid * ROWS_PER_SUB + k * 128, 128)])
        plsc.subcore_barrier()

        pltpu.async_copy(h_hbm.at[srcv.at[0]], rows.at[0], gsem)
        if first:
            fire_w(0, 0)

        def outer(g, _):
            for par in range(2):
                b = g * 2 + par
                buf = rows.at[par]
                pltpu.make_async_copy(h_hbm.at[srcv.at[b]], buf, gsem).wait()
                if first:
                    wait_w(b, par)

                    @pl.when(b + 1 < RPW)
                    def _():
                        fire_w(b + 1, 1 - par)

                    for gq in range(EB // 16):
                        sl = pl.ds(gq * 16, 16)
                        a = als[par, sl] + ard[par, sl]
                        e = jnp.exp(jnp.abs(a) * -2.0)
                        t = (1.0 - e) / (1.0 + e)
                        wv[b, sl] = dis[par, sl] * did[par, sl] * jnp.sign(a) * t

                def sgroup(gq, _):
                    w16 = wv[b, pl.ds(gq * 16, 16)]
                    for k in range(16):
                        i = gq * 16 + k
                        wi = w16[k]
                        for j in range(CW // 16):
                            sl = pl.ds(j * 16, 16)
                            buf[i, sl] = buf[i, sl] * wi
                    return 0

                lax.fori_loop(0, EB // 16, sgroup, 0)

                @pl.when(b > 0)
                def _():
                    pltpu.make_async_copy(
                        rows.at[1 - par], acc.at[dstv.at[b - 1]], ssem).wait()

                pltpu.async_copy(buf, acc.at[dstv.at[b]], ssem, add=True)

                @pl.when(b + 1 < RPW)
                def _():
                    pltpu.async_copy(h_hbm.at[srcv.at[b + 1]], rows.at[1 - par], gsem)
            return 0

        lax.fori_loop(0, RPW // 2, outer, 0)
        pltpu.make_async_copy(rows.at[1], acc.at[dstv.at[RPW - 1]], ssem).wait()
        plsc.subcore_barrier()
        for k in range(ROWS_PER_SUB // 128):
            sl = pl.ds(sid * ROWS_PER_SUB + k * 128, 128)
            pltpu.async_copy(acc.at[sl], o_hbm.at[sl], rsem)
        for k in range(ROWS_PER_SUB // 128):
            sl = pl.ds(sid * ROWS_PER_SUB + k * 128, 128)
            pltpu.make_async_copy(acc.at[sl], o_hbm.at[sl], rsem).wait()
        plsc.subcore_barrier()

    @pl.when(cc == 0)
    def _():
        chunk(h0b, o0, True)
        chunk(h1b, o1, False)
        chunk(h2b, o2, False)
        chunk(h3b, o3, False)

    @pl.when(cc == 1)
    def _():
        chunk(h4b, o4, True)
        chunk(h5b, o5, False)
        chunk(h6b, o6, False)
        chunk(h7b, o7, False)


def _dinv16(deg):
    return jnp.where(deg > 0, lax.rsqrt(jnp.maximum(deg, 1e-30)), 0.0)


def _tc_prep_body(x_ref, w_ref, b_ref, deg_ref, att_ref, *rest):
    hrefs, t_ref = rest[:C], rest[C]
    h = jnp.dot(x_ref[...], w_ref[...], preferred_element_type=jnp.float32)
    h = h + b_ref[...]
    for cix, hr in enumerate(hrefs):
        hr[...] = h[:, cix * CW:(cix + 1) * CW]
    t_ref[...] = (jnp.dot(h, att_ref[...], preferred_element_type=jnp.float32)
                  + _dinv16(deg_ref[...]))


def _tc_mid_body(*refs):
    ocs, pcs = refs[:C], refs[C:2 * C]
    att_ref, deg_ref = refs[2 * C], refs[2 * C + 1]
    hrefs, t_ref = refs[2 * C + 2:3 * C + 2], refs[3 * C + 2]
    hcs = []
    for o, p, hr in zip(ocs, pcs, hrefs):
        hc = o[...] + EPS * p[...]
        hr[...] = hc
        hcs.append(hc)
    h = jnp.concatenate(hcs, axis=1)
    t_ref[...] = (jnp.dot(h, att_ref[...], preferred_element_type=jnp.float32)
                  + _dinv16(deg_ref[...]))


def _tc_final_body(*refs):
    ocs, pcs = refs[:C], refs[C:2 * C]
    w_ref, b_ref, y_ref = refs[2 * C], refs[2 * C + 1], refs[2 * C + 2]
    h = jnp.concatenate(
        [o[...] + EPS * p[...] for o, p in zip(ocs, pcs)], axis=1)
    y_ref[...] = (jnp.dot(h, w_ref[...], preferred_element_type=jnp.float32)
                  + b_ref[...])


def _row_spec(d):
    return pl.BlockSpec((RBLK, d), lambda i: (i, 0))


def _full_spec(r, d):
    return pl.BlockSpec((r, d), lambda i: (0, 0))


_tc_prep = pl.pallas_call(
    _tc_prep_body,
    grid=(GRID,),
    in_specs=[_row_spec(IN), _full_spec(IN, H), _full_spec(1, H),
              _row_spec(16), _full_spec(H, 16)],
    out_specs=[_row_spec(CW)] * C + [_row_spec(16)],
    out_shape=[jax.ShapeDtypeStruct((NP, CW), jnp.float32) for _ in range(C)]
    + [jax.ShapeDtypeStruct((NP, 16), jnp.float32)],
)

_tc_mid = pl.pallas_call(
    _tc_mid_body,
    grid=(GRID,),
    in_specs=[_row_spec(CW)] * (2 * C) + [_full_spec(H, 16), _row_spec(16)],
    out_specs=[_row_spec(CW)] * C + [_row_spec(16)],
    out_shape=[jax.ShapeDtypeStruct((NP, CW), jnp.float32) for _ in range(C)]
    + [jax.ShapeDtypeStruct((NP, 16), jnp.float32)],
)

_tc_final = pl.pallas_call(
    _tc_final_body,
    grid=(GRID,),
    in_specs=[_row_spec(CW)] * (2 * C) + [_full_spec(H, OUT), _full_spec(1, OUT)],
    out_specs=_row_spec(OUT),
    out_shape=jax.ShapeDtypeStruct((NP, OUT), jnp.float32),
)


def kernel(x, edge_index, W_in, b_in, att_l, att_r, W_out, b_out):
    loop = jnp.arange(N, dtype=jnp.int32)
    src = jnp.concatenate([edge_index[0], loop])
    dst = jnp.concatenate([edge_index[1], loop])
    pad = jnp.full((EP - E - N,), PAD_NODE, dtype=jnp.int32)
    src2d = jnp.concatenate([src, pad]).reshape(RB, EB)
    dst2d = jnp.concatenate([dst, pad]).reshape(RB, EB)
    xp = jnp.pad(x, ((0, NP - N), (0, 0)))
    att_big = [
        jnp.zeros((H, 16), jnp.float32)
        .at[:, 0].set(att_l[l]).at[:, 1].set(att_r[l])
        for l in range(L)
    ]
    b_in2 = b_in[None, :]
    b_out2 = b_out[None, :]

    deg16 = _sc_degree(dst2d)
    *h0c, t = _tc_prep(xp, W_in, b_in2, deg16, att_big[0])
    hc = list(h0c)
    oc = None
    for l in range(L):
        al1, ar1, di1 = t[:, 0], t[:, 1], t[:, 2]
        oc = _sc_spmm(src2d, dst2d, al1, ar1, di1, *hc)
        if l + 1 < L:
            *hc, t = _tc_mid(*oc, *h0c, att_big[l + 1], deg16)
    y = _tc_final(*oc, *h0c, W_out, b_out2)
    return y[:N]
